# Initial kernel scaffold; baseline (speedup 1.0000x reference)
import jax
import jax.numpy as jnp
from jax.experimental import pallas as pl


def kernel(x, edge_index, We, be, t0, W1_0, b1_0, g1_0, beta1_0, W2_0, b2_0, t1, W1_1, b1_1, g1_1, beta1_1, W2_1, b2_1, ln1_g, ln1_b, ln0_g, ln0_b, Wl, bl):
    raise NotImplementedError("write your pallas kernel here")


# speedup vs baseline: 16.8029x; 16.8029x over previous
import functools

import jax
import jax.numpy as jnp
from jax import lax
from jax.experimental import pallas as pl
from jax.experimental.pallas import tpu as pltpu
from jax.experimental.pallas import tpu_sc as plsc

N = 10000
E = 320000
D = 128
H = 64

_BLK = 1000
_GRID = N // _BLK

_CW = 125
_CNW = E // 16 // _CW
_RB = 640
_RZB = 80

_NW = 80
_NNW = 128
_E2 = 32 * _NNW * _NW

_MESH = plsc.VectorSubcoreMesh(core_axis_name="c", subcore_axis_name="s")
_SC_PARAMS = pltpu.CompilerParams(use_tc_tiling_on_sc=False)
_SC_PARAMS_NOLAYOUT = pltpu.CompilerParams(
    use_tc_tiling_on_sc=False, needs_layout_passes=False)


def _ln(h, g, b):
    mu = jnp.mean(h, axis=-1, keepdims=True)
    var = jnp.mean((h - mu) ** 2, axis=-1, keepdims=True)
    return (h - mu) / jnp.sqrt(var + 1e-5) * g + b


def _tab(h, t):
    m = jnp.maximum(h, 0.0) + 1e-7
    a = jnp.exp(t * m)
    return a, m * a


def _tc_pre_body(x_ref, we_ref, be_ref, t_ref, h0_ref, tab_ref):
    h0 = jnp.dot(x_ref[...], we_ref[...], preferred_element_type=jnp.float32)
    h0 = h0 + be_ref[0:1, :]
    h0_ref[...] = h0
    a, b = _tab(h0, t_ref[0, 0])
    tab_ref[0] = a
    tab_ref[1] = b


def _tc_pre(x, we, be, t):
    return pl.pallas_call(
        _tc_pre_body,
        grid=(_GRID,),
        in_specs=[
            pl.BlockSpec((_BLK, D), lambda i: (i, 0)),
            pl.BlockSpec((D, H), lambda i: (0, 0)),
            pl.BlockSpec((8, H), lambda i: (0, 0)),
            pl.BlockSpec((8, 128), lambda i: (0, 0)),
        ],
        out_specs=[
            pl.BlockSpec((_BLK, H), lambda i: (i, 0)),
            pl.BlockSpec((2, _BLK, H), lambda i: (0, i, 0)),
        ],
        out_shape=[
            jax.ShapeDtypeStruct((N, H), jnp.float32),
            jax.ShapeDtypeStruct((2, N, H), jnp.float32),
        ],
    )(x, we, be, t)


def _tc_mid_body(h0_ref, dn_ref, w1_ref, b1_ref, g1_ref, be1_ref, w2_ref,
                 b2_ref, lng_ref, lnb_ref, t_ref, x1_ref, hr_ref, tab_ref):
    den = dn_ref[0]
    num = dn_ref[1]
    out0 = num / (den + 1e-16) + h0_ref[...]
    hh = jnp.dot(out0, w1_ref[...], preferred_element_type=jnp.float32)
    hh = hh + b1_ref[0:1, :]
    hh = jnp.maximum(_ln(hh, g1_ref[0:1, :], be1_ref[0:1, :]), 0.0)
    x1 = jnp.dot(hh, w2_ref[...], preferred_element_type=jnp.float32)
    x1 = x1 + b2_ref[0:1, :]
    x1_ref[...] = x1
    hr = jnp.maximum(_ln(x1, lng_ref[0:1, :], lnb_ref[0:1, :]), 0.0)
    hr_ref[...] = hr
    a, b = _tab(hr, t_ref[0, 0])
    tab_ref[0] = a
    tab_ref[1] = b


def _tc_mid(h0, dn, w1, b1, g1, be1, w2, b2, lng, lnb, t):
    return pl.pallas_call(
        _tc_mid_body,
        grid=(_GRID,),
        in_specs=[
            pl.BlockSpec((_BLK, H), lambda i: (i, 0)),
            pl.BlockSpec((2, _BLK, H), lambda i: (0, i, 0)),
            pl.BlockSpec((H, 2 * H), lambda i: (0, 0)),
            pl.BlockSpec((8, 2 * H), lambda i: (0, 0)),
            pl.BlockSpec((8, 2 * H), lambda i: (0, 0)),
            pl.BlockSpec((8, 2 * H), lambda i: (0, 0)),
            pl.BlockSpec((2 * H, H), lambda i: (0, 0)),
            pl.BlockSpec((8, H), lambda i: (0, 0)),
            pl.BlockSpec((8, H), lambda i: (0, 0)),
            pl.BlockSpec((8, H), lambda i: (0, 0)),
            pl.BlockSpec((8, 128), lambda i: (0, 0)),
        ],
        out_specs=[
            pl.BlockSpec((_BLK, H), lambda i: (i, 0)),
            pl.BlockSpec((_BLK, H), lambda i: (i, 0)),
            pl.BlockSpec((2, _BLK, H), lambda i: (0, i, 0)),
        ],
        out_shape=[
            jax.ShapeDtypeStruct((N, H), jnp.float32),
            jax.ShapeDtypeStruct((N, H), jnp.float32),
            jax.ShapeDtypeStruct((2, N, H), jnp.float32),
        ],
    )(h0, dn, w1, b1, g1, be1, w2, b2, lng, lnb, t)


def _tc_post_body(x1_ref, hr_ref, dn_ref, w1_ref, b1_ref, g1_ref, be1_ref,
                  w2_ref, b2_ref, lng_ref, lnb_ref, wl_ref, bl_ref,
                  logit_ref):
    den = dn_ref[0]
    num = dn_ref[1]
    out1 = num / (den + 1e-16) + hr_ref[...]
    hh = jnp.dot(out1, w1_ref[...], preferred_element_type=jnp.float32)
    hh = hh + b1_ref[0:1, :]
    hh = jnp.maximum(_ln(hh, g1_ref[0:1, :], be1_ref[0:1, :]), 0.0)
    h2 = jnp.dot(hh, w2_ref[...], preferred_element_type=jnp.float32)
    h2 = h2 + b2_ref[0:1, :]
    x2 = x1_ref[...] + h2
    hf = jnp.maximum(_ln(x2, lng_ref[0:1, :], lnb_ref[0:1, :]), 0.0)
    logit_ref[...] = (
        jnp.dot(hf, wl_ref[...], preferred_element_type=jnp.float32)
        + bl_ref[0, 0]
    )


def _tc_post(x1, hr, dn, w1, b1, g1, be1, w2, b2, lng, lnb, wl, bl):
    return pl.pallas_call(
        _tc_post_body,
        grid=(_GRID,),
        in_specs=[
            pl.BlockSpec((_BLK, H), lambda i: (i, 0)),
            pl.BlockSpec((_BLK, H), lambda i: (i, 0)),
            pl.BlockSpec((2, _BLK, H), lambda i: (0, i, 0)),
            pl.BlockSpec((H, 2 * H), lambda i: (0, 0)),
            pl.BlockSpec((8, 2 * H), lambda i: (0, 0)),
            pl.BlockSpec((8, 2 * H), lambda i: (0, 0)),
            pl.BlockSpec((8, 2 * H), lambda i: (0, 0)),
            pl.BlockSpec((2 * H, H), lambda i: (0, 0)),
            pl.BlockSpec((8, H), lambda i: (0, 0)),
            pl.BlockSpec((8, H), lambda i: (0, 0)),
            pl.BlockSpec((8, H), lambda i: (0, 0)),
            pl.BlockSpec((H, 1), lambda i: (0, 0)),
            pl.BlockSpec((8, 128), lambda i: (0, 0)),
        ],
        out_specs=pl.BlockSpec((_BLK, 1), lambda i: (i, 0)),
        out_shape=jax.ShapeDtypeStruct((N, 1), jnp.float32),
    )(x1, hr, dn, w1, b1, g1, be1, w2, b2, lng, lnb, wl, bl)


def _sc_conv(tab, tsrc2, dst2):
    @functools.partial(
        pl.kernel,
        out_type=jax.ShapeDtypeStruct((2 * N, H), jnp.float32),
        mesh=_MESH,
        scratch_types=[
            pltpu.VMEM((_CNW, _CW), jnp.int32),
            pltpu.VMEM((_CNW, _CW), jnp.int32),
            pltpu.VMEM((_CW, H), jnp.float32),
            pltpu.VMEM((_RZB, H), jnp.float32),
            pltpu.VMEM_SHARED((N, H), jnp.float32),
            pltpu.SemaphoreType.DMA,
        ],
        compiler_params=_SC_PARAMS,
    )
    def k(tab_h, src_h, dst_h, out_h, src_v, dst_v, rows_v, zb_v, acc_sh,
          sem):
        c = lax.axis_index("c")
        s = lax.axis_index("s")
        zero = jnp.zeros((16,), jnp.float32)

        @pl.loop(0, _RZB)
        def _(i):
            zb_v[i, pl.ds(0, 16)] = zero
            zb_v[i, pl.ds(16, 16)] = zero
            zb_v[i, pl.ds(32, 16)] = zero
            zb_v[i, pl.ds(48, 16)] = zero

        rb = s * _RB
        ncp = jnp.where(s == 15, 5, 8)

        def zcp(i, carry):
            pltpu.sync_copy(zb_v, acc_sh.at[pl.ds(rb + i * _RZB, _RZB)])
            return carry

        lax.fori_loop(0, ncp, zcp, 0)

        pltpu.sync_copy(src_h.at[pl.ds((c * 16 + s) * _CNW, _CNW)], src_v)
        pltpu.sync_copy(dst_h.at[pl.ds(s * _CNW, _CNW)], dst_v)

        plsc.subcore_barrier()

        @pl.loop(0, _CNW)
        def _(j):
            pltpu.sync_copy(tab_h.at[src_v.at[j]], rows_v)
            pltpu.sync_copy(rows_v, acc_sh.at[dst_v.at[j]], add=True)

        plsc.subcore_barrier()

        def wcp(i, carry):
            pltpu.sync_copy(
                acc_sh.at[pl.ds(rb + i * _RZB, _RZB)],
                out_h.at[pl.ds(c * N + rb + i * _RZB, _RZB)],
            )
            return carry

        lax.fori_loop(0, ncp, wcp, 0)

    return k(tab, tsrc2, dst2)


def _sc_nmax(lg, src3, dst3):
    @functools.partial(
        pl.kernel,
        out_type=jax.ShapeDtypeStruct((2 * N, 16), jnp.float32),
        mesh=_MESH,
        scratch_types=[
            pltpu.VMEM((_NNW, _NW), jnp.int32),
            pltpu.VMEM((_NNW, _NW), jnp.int32),
            pltpu.VMEM((N,), jnp.float32),
            pltpu.VMEM((_NW, 16), jnp.float32),
            pltpu.VMEM((_RZB, 16), jnp.float32),
            pltpu.VMEM_SHARED((N, 16), jnp.float32),
        ],
        compiler_params=_SC_PARAMS_NOLAYOUT,
    )
    def k(lg_h, src_h, dst_h, out_h, src_v, dst_v, lg_v, col_v, zb_v,
          cnt_sh):
        c = lax.axis_index("c")
        s = lax.axis_index("s")
        zero = jnp.zeros((16,), jnp.float32)
        lanes = lax.iota(jnp.int32, 16)
        zlane = jnp.zeros((16,), jnp.int32)

        @pl.loop(0, _RZB)
        def _(i):
            zb_v[i, pl.ds(0, 16)] = zero

        @pl.loop(0, _NW)
        def _(i):
            col_v[i, pl.ds(0, 16)] = zero

        rb = s * _RB
        ncp = jnp.where(s == 15, 5, 8)

        def zcp(i, carry):
            pltpu.sync_copy(zb_v, cnt_sh.at[pl.ds(rb + i * _RZB, _RZB)])
            return carry

        lax.fori_loop(0, ncp, zcp, 0)

        pltpu.sync_copy(lg_h, lg_v)
        pltpu.sync_copy(src_h.at[pl.ds((c * 16 + s) * _NNW, _NNW)], src_v)
        pltpu.sync_copy(dst_h.at[pl.ds((c * 16 + s) * _NNW, _NNW)], dst_v)

        plsc.subcore_barrier()

        @pl.loop(0, _NNW)
        def _(j):
            @pl.loop(0, _NW // 16)
            def _(g):
                sv = src_v[j, pl.ds(g * 16, 16)]
                dv = dst_v[j, pl.ds(g * 16, 16)]
                ls = plsc.load_gather(lg_v, [sv])
                ld = plsc.load_gather(lg_v, [dv])
                ind = jnp.where(ls > ld, 1.0, 0.0).astype(jnp.float32)
                plsc.store_scatter(col_v, [g * 16 + lanes, zlane], ind)

            pltpu.sync_copy(col_v, cnt_sh.at[dst_v.at[j]], add=True)

        plsc.subcore_barrier()

        def wcp(i, carry):
            pltpu.sync_copy(
                cnt_sh.at[pl.ds(rb + i * _RZB, _RZB)],
                out_h.at[pl.ds(c * N + rb + i * _RZB, _RZB)],
            )
            return carry

        lax.fori_loop(0, ncp, wcp, 0)

    return k(lg, src3, dst3)


def _tc_final_body(logit_ref, cnt_ref, out_ref):
    tot = cnt_ref[0, :, 0:1] + cnt_ref[1, :, 0:1]
    out_ref[...] = jnp.where(tot > 0.0, 0.0, 1.0)


def _tc_final(logits, cnt):
    return pl.pallas_call(
        _tc_final_body,
        grid=(_GRID,),
        in_specs=[
            pl.BlockSpec((_BLK, 1), lambda i: (i, 0)),
            pl.BlockSpec((2, _BLK, 16), lambda i: (0, i, 0)),
        ],
        out_specs=pl.BlockSpec((_BLK, 1), lambda i: (i, 0)),
        out_shape=jax.ShapeDtypeStruct((N, 1), jnp.float32),
    )(logits, cnt)


def kernel(x, edge_index, We, be, t0, W1_0, b1_0, g1_0, beta1_0, W2_0, b2_0,
           t1, W1_1, b1_1, g1_1, beta1_1, W2_1, b2_1, ln1_g, ln1_b, ln0_g,
           ln0_b, Wl, bl):
    f32 = jnp.float32

    def r2(v, d):
        return jnp.broadcast_to(v.reshape(1, d).astype(f32), (8, d))

    src = edge_index[0]
    dst = edge_index[1]
    tsrc2 = jnp.concatenate([src, src + N]).reshape(2 * E // _CW, _CW)
    dst2 = dst.reshape(E // _CW, _CW)
    pad = jnp.zeros((_E2 - E,), jnp.int32)
    src3 = jnp.concatenate([src, pad]).reshape(_E2 // _NW, _NW)
    dst3 = jnp.concatenate([dst, pad]).reshape(_E2 // _NW, _NW)

    t0b = jnp.broadcast_to(t0.reshape(1, 1).astype(f32), (8, 128))
    t1b = jnp.broadcast_to(t1.reshape(1, 1).astype(f32), (8, 128))

    h0, tab0 = _tc_pre(x, We, r2(be, H), t0b)
    dn0 = _sc_conv(tab0.reshape(2 * N, H), tsrc2, dst2).reshape(2, N, H)
    x1, hr, tab1 = _tc_mid(h0, dn0, W1_0, r2(b1_0, 2 * H), r2(g1_0, 2 * H),
                           r2(beta1_0, 2 * H), W2_0, r2(b2_0, H),
                           r2(ln1_g, H), r2(ln1_b, H), t1b)
    dn1 = _sc_conv(tab1.reshape(2 * N, H), tsrc2, dst2).reshape(2, N, H)
    logits = _tc_post(x1, hr, dn1, W1_1, r2(b1_1, 2 * H), r2(g1_1, 2 * H),
                      r2(beta1_1, 2 * H), W2_1, r2(b2_1, H), r2(ln0_g, H),
                      r2(ln0_b, H), Wl,
                      jnp.broadcast_to(bl.reshape(1, 1).astype(f32),
                                       (8, 128)))
    cnt = _sc_nmax(logits.reshape(N), src3, dst3).reshape(2, N, 16)
    out = _tc_final(logits, cnt)
    return (out, logits)


# speedup vs baseline: 20.9656x; 1.2477x over previous
import functools

import jax
import jax.numpy as jnp
from jax import lax
from jax.experimental import pallas as pl
from jax.experimental.pallas import tpu as pltpu
from jax.experimental.pallas import tpu_sc as plsc

N = 10000
E = 320000
D = 128
H = 64

_BLK = 1000
_GRID = N // _BLK

_CW = 125
_CNW = E // 16 // _CW
_RB = 640
_RZB = 80

_NW = 80
_NNW = 128
_E2 = 32 * _NNW * _NW

_MESH = plsc.VectorSubcoreMesh(core_axis_name="c", subcore_axis_name="s")
_SC_PARAMS = pltpu.CompilerParams(use_tc_tiling_on_sc=False)
_SC_PARAMS_NOLAYOUT = pltpu.CompilerParams(
    use_tc_tiling_on_sc=False, needs_layout_passes=False)


def _ln(h, g, b):
    mu = jnp.mean(h, axis=-1, keepdims=True)
    var = jnp.mean((h - mu) ** 2, axis=-1, keepdims=True)
    return (h - mu) / jnp.sqrt(var + 1e-5) * g + b


def _tab(h, t):
    m = jnp.maximum(h, 0.0) + 1e-7
    a = jnp.exp(t * m)
    return a, m * a


def _tc_pre_body(x_ref, we_ref, be_ref, t_ref, h0_ref, tab_ref):
    h0 = jnp.dot(x_ref[...], we_ref[...], preferred_element_type=jnp.float32)
    h0 = h0 + be_ref[0:1, :]
    h0_ref[...] = h0
    a, b = _tab(h0, t_ref[0, 0])
    tab_ref[0] = a
    tab_ref[1] = b


def _tc_pre(x, we, be, t):
    return pl.pallas_call(
        _tc_pre_body,
        grid=(_GRID,),
        in_specs=[
            pl.BlockSpec((_BLK, D), lambda i: (i, 0)),
            pl.BlockSpec((D, H), lambda i: (0, 0)),
            pl.BlockSpec((8, H), lambda i: (0, 0)),
            pl.BlockSpec((8, 128), lambda i: (0, 0)),
        ],
        out_specs=[
            pl.BlockSpec((_BLK, H), lambda i: (i, 0)),
            pl.BlockSpec((2, _BLK, H), lambda i: (0, i, 0)),
        ],
        out_shape=[
            jax.ShapeDtypeStruct((N, H), jnp.float32),
            jax.ShapeDtypeStruct((2, N, H), jnp.float32),
        ],
    )(x, we, be, t)


def _tc_mid_body(h0_ref, dn_ref, w1_ref, b1_ref, g1_ref, be1_ref, w2_ref,
                 b2_ref, lng_ref, lnb_ref, t_ref, x1_ref, hr_ref, tab_ref):
    den = dn_ref[0]
    num = dn_ref[1]
    out0 = num / (den + 1e-16) + h0_ref[...]
    hh = jnp.dot(out0, w1_ref[...], preferred_element_type=jnp.float32)
    hh = hh + b1_ref[0:1, :]
    hh = jnp.maximum(_ln(hh, g1_ref[0:1, :], be1_ref[0:1, :]), 0.0)
    x1 = jnp.dot(hh, w2_ref[...], preferred_element_type=jnp.float32)
    x1 = x1 + b2_ref[0:1, :]
    x1_ref[...] = x1
    hr = jnp.maximum(_ln(x1, lng_ref[0:1, :], lnb_ref[0:1, :]), 0.0)
    hr_ref[...] = hr
    a, b = _tab(hr, t_ref[0, 0])
    tab_ref[0] = a
    tab_ref[1] = b


def _tc_mid(h0, dn, w1, b1, g1, be1, w2, b2, lng, lnb, t):
    return pl.pallas_call(
        _tc_mid_body,
        grid=(_GRID,),
        in_specs=[
            pl.BlockSpec((_BLK, H), lambda i: (i, 0)),
            pl.BlockSpec((2, _BLK, H), lambda i: (0, i, 0)),
            pl.BlockSpec((H, 2 * H), lambda i: (0, 0)),
            pl.BlockSpec((8, 2 * H), lambda i: (0, 0)),
            pl.BlockSpec((8, 2 * H), lambda i: (0, 0)),
            pl.BlockSpec((8, 2 * H), lambda i: (0, 0)),
            pl.BlockSpec((2 * H, H), lambda i: (0, 0)),
            pl.BlockSpec((8, H), lambda i: (0, 0)),
            pl.BlockSpec((8, H), lambda i: (0, 0)),
            pl.BlockSpec((8, H), lambda i: (0, 0)),
            pl.BlockSpec((8, 128), lambda i: (0, 0)),
        ],
        out_specs=[
            pl.BlockSpec((_BLK, H), lambda i: (i, 0)),
            pl.BlockSpec((_BLK, H), lambda i: (i, 0)),
            pl.BlockSpec((2, _BLK, H), lambda i: (0, i, 0)),
        ],
        out_shape=[
            jax.ShapeDtypeStruct((N, H), jnp.float32),
            jax.ShapeDtypeStruct((N, H), jnp.float32),
            jax.ShapeDtypeStruct((2, N, H), jnp.float32),
        ],
    )(h0, dn, w1, b1, g1, be1, w2, b2, lng, lnb, t)


def _tc_post_body(x1_ref, hr_ref, dn_ref, w1_ref, b1_ref, g1_ref, be1_ref,
                  w2_ref, b2_ref, lng_ref, lnb_ref, wl_ref, bl_ref,
                  logit_ref):
    den = dn_ref[0]
    num = dn_ref[1]
    out1 = num / (den + 1e-16) + hr_ref[...]
    hh = jnp.dot(out1, w1_ref[...], preferred_element_type=jnp.float32)
    hh = hh + b1_ref[0:1, :]
    hh = jnp.maximum(_ln(hh, g1_ref[0:1, :], be1_ref[0:1, :]), 0.0)
    h2 = jnp.dot(hh, w2_ref[...], preferred_element_type=jnp.float32)
    h2 = h2 + b2_ref[0:1, :]
    x2 = x1_ref[...] + h2
    hf = jnp.maximum(_ln(x2, lng_ref[0:1, :], lnb_ref[0:1, :]), 0.0)
    logit_ref[...] = (
        jnp.dot(hf, wl_ref[...], preferred_element_type=jnp.float32)
        + bl_ref[0, 0]
    )


def _tc_post(x1, hr, dn, w1, b1, g1, be1, w2, b2, lng, lnb, wl, bl):
    return pl.pallas_call(
        _tc_post_body,
        grid=(_GRID,),
        in_specs=[
            pl.BlockSpec((_BLK, H), lambda i: (i, 0)),
            pl.BlockSpec((_BLK, H), lambda i: (i, 0)),
            pl.BlockSpec((2, _BLK, H), lambda i: (0, i, 0)),
            pl.BlockSpec((H, 2 * H), lambda i: (0, 0)),
            pl.BlockSpec((8, 2 * H), lambda i: (0, 0)),
            pl.BlockSpec((8, 2 * H), lambda i: (0, 0)),
            pl.BlockSpec((8, 2 * H), lambda i: (0, 0)),
            pl.BlockSpec((2 * H, H), lambda i: (0, 0)),
            pl.BlockSpec((8, H), lambda i: (0, 0)),
            pl.BlockSpec((8, H), lambda i: (0, 0)),
            pl.BlockSpec((8, H), lambda i: (0, 0)),
            pl.BlockSpec((H, 1), lambda i: (0, 0)),
            pl.BlockSpec((8, 128), lambda i: (0, 0)),
        ],
        out_specs=pl.BlockSpec((_BLK, 1), lambda i: (i, 0)),
        out_shape=jax.ShapeDtypeStruct((N, 1), jnp.float32),
    )(x1, hr, dn, w1, b1, g1, be1, w2, b2, lng, lnb, wl, bl)


def _sc_conv(tab, tsrc2, dst2):
    @functools.partial(
        pl.kernel,
        out_type=jax.ShapeDtypeStruct((2 * N, H), jnp.float32),
        mesh=_MESH,
        scratch_types=[
            pltpu.VMEM((_CNW, _CW), jnp.int32),
            pltpu.VMEM((_CNW, _CW), jnp.int32),
            pltpu.VMEM((_CW, H), jnp.float32),
            pltpu.VMEM((_CW, H), jnp.float32),
            pltpu.VMEM((_RZB, H), jnp.float32),
            pltpu.VMEM_SHARED((N, H), jnp.float32),
            pltpu.SemaphoreType.DMA,
            pltpu.SemaphoreType.DMA,
            pltpu.SemaphoreType.DMA,
            pltpu.SemaphoreType.DMA,
        ],
        compiler_params=_SC_PARAMS,
    )
    def k(tab_h, src_h, dst_h, out_h, src_v, dst_v, ra_v, rb_v, zb_v, acc_sh,
          sga, sgb, ssa, ssb):
        c = lax.axis_index("c")
        s = lax.axis_index("s")
        zero = jnp.zeros((16,), jnp.float32)

        @pl.loop(0, _RZB)
        def _(i):
            zb_v[i, pl.ds(0, 16)] = zero
            zb_v[i, pl.ds(16, 16)] = zero
            zb_v[i, pl.ds(32, 16)] = zero
            zb_v[i, pl.ds(48, 16)] = zero

        rb = s * _RB
        ncp = jnp.where(s == 15, 5, 8)

        def zcp(i, carry):
            pltpu.sync_copy(zb_v, acc_sh.at[pl.ds(rb + i * _RZB, _RZB)])
            return carry

        lax.fori_loop(0, ncp, zcp, 0)

        pltpu.sync_copy(src_h.at[pl.ds((c * 16 + s) * _CNW, _CNW)], src_v)
        pltpu.sync_copy(dst_h.at[pl.ds(s * _CNW, _CNW)], dst_v)

        plsc.subcore_barrier()

        pltpu.async_copy(tab_h.at[src_v.at[0]], ra_v, sga)
        pltpu.async_copy(tab_h.at[src_v.at[1]], rb_v, sgb)

        @pl.loop(0, _CNW - 2, step=2)
        def _(j):
            pltpu.make_async_copy(tab_h.at[src_v.at[j]], ra_v, sga).wait()
            pltpu.async_copy(ra_v, acc_sh.at[dst_v.at[j]], ssa, add=True)
            pltpu.make_async_copy(
                tab_h.at[src_v.at[j + 1]], rb_v, sgb).wait()
            pltpu.async_copy(rb_v, acc_sh.at[dst_v.at[j + 1]], ssb, add=True)
            pltpu.make_async_copy(ra_v, acc_sh.at[dst_v.at[j]], ssa).wait()
            pltpu.async_copy(tab_h.at[src_v.at[j + 2]], ra_v, sga)
            pltpu.make_async_copy(
                rb_v, acc_sh.at[dst_v.at[j + 1]], ssb).wait()
            pltpu.async_copy(tab_h.at[src_v.at[j + 3]], rb_v, sgb)

        j0 = _CNW - 2
        pltpu.make_async_copy(tab_h.at[src_v.at[j0]], ra_v, sga).wait()
        pltpu.async_copy(ra_v, acc_sh.at[dst_v.at[j0]], ssa, add=True)
        pltpu.make_async_copy(tab_h.at[src_v.at[j0 + 1]], rb_v, sgb).wait()
        pltpu.async_copy(rb_v, acc_sh.at[dst_v.at[j0 + 1]], ssb, add=True)
        pltpu.make_async_copy(ra_v, acc_sh.at[dst_v.at[j0]], ssa).wait()
        pltpu.make_async_copy(rb_v, acc_sh.at[dst_v.at[j0 + 1]], ssb).wait()

        plsc.subcore_barrier()

        def wcp(i, carry):
            pltpu.sync_copy(
                acc_sh.at[pl.ds(rb + i * _RZB, _RZB)],
                out_h.at[pl.ds(c * N + rb + i * _RZB, _RZB)],
            )
            return carry

        lax.fori_loop(0, ncp, wcp, 0)

    return k(tab, tsrc2, dst2)


def _sc_nmax(lg, src3, dst3):
    @functools.partial(
        pl.kernel,
        out_type=jax.ShapeDtypeStruct((2 * N, 16), jnp.float32),
        mesh=_MESH,
        scratch_types=[
            pltpu.VMEM((_NNW, _NW), jnp.int32),
            pltpu.VMEM((_NNW, _NW), jnp.int32),
            pltpu.VMEM((N,), jnp.float32),
            pltpu.VMEM((_NW, 16), jnp.float32),
            pltpu.VMEM((_RZB, 16), jnp.float32),
            pltpu.VMEM_SHARED((N, 16), jnp.float32),
        ],
        compiler_params=_SC_PARAMS_NOLAYOUT,
    )
    def k(lg_h, src_h, dst_h, out_h, src_v, dst_v, lg_v, col_v, zb_v,
          cnt_sh):
        c = lax.axis_index("c")
        s = lax.axis_index("s")
        zero = jnp.zeros((16,), jnp.float32)
        lanes = lax.iota(jnp.int32, 16)
        zlane = jnp.zeros((16,), jnp.int32)

        @pl.loop(0, _RZB)
        def _(i):
            zb_v[i, pl.ds(0, 16)] = zero

        @pl.loop(0, _NW)
        def _(i):
            col_v[i, pl.ds(0, 16)] = zero

        rb = s * _RB
        ncp = jnp.where(s == 15, 5, 8)

        def zcp(i, carry):
            pltpu.sync_copy(zb_v, cnt_sh.at[pl.ds(rb + i * _RZB, _RZB)])
            return carry

        lax.fori_loop(0, ncp, zcp, 0)

        pltpu.sync_copy(lg_h, lg_v)
        pltpu.sync_copy(src_h.at[pl.ds((c * 16 + s) * _NNW, _NNW)], src_v)
        pltpu.sync_copy(dst_h.at[pl.ds((c * 16 + s) * _NNW, _NNW)], dst_v)

        plsc.subcore_barrier()

        @pl.loop(0, _NNW)
        def _(j):
            @pl.loop(0, _NW // 16)
            def _(g):
                sv = src_v[j, pl.ds(g * 16, 16)]
                dv = dst_v[j, pl.ds(g * 16, 16)]
                ls = plsc.load_gather(lg_v, [sv])
                ld = plsc.load_gather(lg_v, [dv])
                ind = jnp.where(ls > ld, 1.0, 0.0).astype(jnp.float32)
                plsc.store_scatter(col_v, [g * 16 + lanes, zlane], ind)

            pltpu.sync_copy(col_v, cnt_sh.at[dst_v.at[j]], add=True)

        plsc.subcore_barrier()

        def wcp(i, carry):
            pltpu.sync_copy(
                cnt_sh.at[pl.ds(rb + i * _RZB, _RZB)],
                out_h.at[pl.ds(c * N + rb + i * _RZB, _RZB)],
            )
            return carry

        lax.fori_loop(0, ncp, wcp, 0)

    return k(lg, src3, dst3)


def _tc_final_body(logit_ref, cnt_ref, out_ref):
    tot = cnt_ref[0, :, 0:1] + cnt_ref[1, :, 0:1]
    out_ref[...] = jnp.where(tot > 0.0, 0.0, 1.0)


def _tc_final(logits, cnt):
    return pl.pallas_call(
        _tc_final_body,
        grid=(_GRID,),
        in_specs=[
            pl.BlockSpec((_BLK, 1), lambda i: (i, 0)),
            pl.BlockSpec((2, _BLK, 16), lambda i: (0, i, 0)),
        ],
        out_specs=pl.BlockSpec((_BLK, 1), lambda i: (i, 0)),
        out_shape=jax.ShapeDtypeStruct((N, 1), jnp.float32),
    )(logits, cnt)


def kernel(x, edge_index, We, be, t0, W1_0, b1_0, g1_0, beta1_0, W2_0, b2_0,
           t1, W1_1, b1_1, g1_1, beta1_1, W2_1, b2_1, ln1_g, ln1_b, ln0_g,
           ln0_b, Wl, bl):
    f32 = jnp.float32

    def r2(v, d):
        return jnp.broadcast_to(v.reshape(1, d).astype(f32), (8, d))

    src = edge_index[0]
    dst = edge_index[1]
    tsrc2 = jnp.concatenate([src, src + N]).reshape(2 * E // _CW, _CW)
    dst2 = dst.reshape(E // _CW, _CW)
    pad = jnp.zeros((_E2 - E,), jnp.int32)
    src3 = jnp.concatenate([src, pad]).reshape(_E2 // _NW, _NW)
    dst3 = jnp.concatenate([dst, pad]).reshape(_E2 // _NW, _NW)

    t0b = jnp.broadcast_to(t0.reshape(1, 1).astype(f32), (8, 128))
    t1b = jnp.broadcast_to(t1.reshape(1, 1).astype(f32), (8, 128))

    h0, tab0 = _tc_pre(x, We, r2(be, H), t0b)
    dn0 = _sc_conv(tab0.reshape(2 * N, H), tsrc2, dst2).reshape(2, N, H)
    x1, hr, tab1 = _tc_mid(h0, dn0, W1_0, r2(b1_0, 2 * H), r2(g1_0, 2 * H),
                           r2(beta1_0, 2 * H), W2_0, r2(b2_0, H),
                           r2(ln1_g, H), r2(ln1_b, H), t1b)
    dn1 = _sc_conv(tab1.reshape(2 * N, H), tsrc2, dst2).reshape(2, N, H)
    logits = _tc_post(x1, hr, dn1, W1_1, r2(b1_1, 2 * H), r2(g1_1, 2 * H),
                      r2(beta1_1, 2 * H), W2_1, r2(b2_1, H), r2(ln0_g, H),
                      r2(ln0_b, H), Wl,
                      jnp.broadcast_to(bl.reshape(1, 1).astype(f32),
                                       (8, 128)))
    cnt = _sc_nmax(logits.reshape(N), src3, dst3).reshape(2, N, 16)
    out = _tc_final(logits, cnt)
    return (out, logits)


# speedup vs baseline: 25.9325x; 1.2369x over previous
import functools

import jax
import jax.numpy as jnp
from jax import lax
from jax.experimental import pallas as pl
from jax.experimental.pallas import tpu as pltpu
from jax.experimental.pallas import tpu_sc as plsc

N = 10000
E = 320000
D = 128
H = 64

_BLK = 1000
_GRID = N // _BLK

_CW = 125
_CNW = E // 16 // _CW
_RB = 640
_RZB = 80

_NW = 80
_NNW = 128
_E2 = 32 * _NNW * _NW

_MESH = plsc.VectorSubcoreMesh(core_axis_name="c", subcore_axis_name="s")
_SC_PARAMS = pltpu.CompilerParams(use_tc_tiling_on_sc=False)
_SC_PARAMS_NOLAYOUT = pltpu.CompilerParams(
    use_tc_tiling_on_sc=False, needs_layout_passes=False)


def _ln(h, g, b):
    mu = jnp.mean(h, axis=-1, keepdims=True)
    var = jnp.mean((h - mu) ** 2, axis=-1, keepdims=True)
    return (h - mu) / jnp.sqrt(var + 1e-5) * g + b


def _tab(h, t):
    m = jnp.maximum(h, 0.0) + 1e-7
    a = jnp.exp(t * m)
    return a, m * a


def _tc_pre_body(x_ref, we_ref, be_ref, t_ref, h0_ref, tab_ref):
    h0 = jnp.dot(x_ref[...], we_ref[...], preferred_element_type=jnp.float32)
    h0 = h0 + be_ref[0:1, :]
    h0_ref[...] = h0
    a, b = _tab(h0, t_ref[0, 0])
    tab_ref[0] = a
    tab_ref[1] = b


def _tc_pre(x, we, be, t):
    return pl.pallas_call(
        _tc_pre_body,
        grid=(_GRID,),
        in_specs=[
            pl.BlockSpec((_BLK, D), lambda i: (i, 0)),
            pl.BlockSpec((D, H), lambda i: (0, 0)),
            pl.BlockSpec((8, H), lambda i: (0, 0)),
            pl.BlockSpec((8, 128), lambda i: (0, 0)),
        ],
        out_specs=[
            pl.BlockSpec((_BLK, H), lambda i: (i, 0)),
            pl.BlockSpec((2, _BLK, H), lambda i: (0, i, 0)),
        ],
        out_shape=[
            jax.ShapeDtypeStruct((N, H), jnp.float32),
            jax.ShapeDtypeStruct((2, N, H), jnp.float32),
        ],
    )(x, we, be, t)


def _tc_mid_body(h0_ref, dn_ref, w1_ref, b1_ref, g1_ref, be1_ref, w2_ref,
                 b2_ref, lng_ref, lnb_ref, t_ref, x1_ref, hr_ref, tab_ref):
    den = dn_ref[0]
    num = dn_ref[1]
    out0 = num / (den + 1e-16) + h0_ref[...]
    hh = jnp.dot(out0, w1_ref[...], preferred_element_type=jnp.float32)
    hh = hh + b1_ref[0:1, :]
    hh = jnp.maximum(_ln(hh, g1_ref[0:1, :], be1_ref[0:1, :]), 0.0)
    x1 = jnp.dot(hh, w2_ref[...], preferred_element_type=jnp.float32)
    x1 = x1 + b2_ref[0:1, :]
    x1_ref[...] = x1
    hr = jnp.maximum(_ln(x1, lng_ref[0:1, :], lnb_ref[0:1, :]), 0.0)
    hr_ref[...] = hr
    a, b = _tab(hr, t_ref[0, 0])
    tab_ref[0] = a
    tab_ref[1] = b


def _tc_mid(h0, dn, w1, b1, g1, be1, w2, b2, lng, lnb, t):
    return pl.pallas_call(
        _tc_mid_body,
        grid=(_GRID,),
        in_specs=[
            pl.BlockSpec((_BLK, H), lambda i: (i, 0)),
            pl.BlockSpec((2, _BLK, H), lambda i: (0, i, 0)),
            pl.BlockSpec((H, 2 * H), lambda i: (0, 0)),
            pl.BlockSpec((8, 2 * H), lambda i: (0, 0)),
            pl.BlockSpec((8, 2 * H), lambda i: (0, 0)),
            pl.BlockSpec((8, 2 * H), lambda i: (0, 0)),
            pl.BlockSpec((2 * H, H), lambda i: (0, 0)),
            pl.BlockSpec((8, H), lambda i: (0, 0)),
            pl.BlockSpec((8, H), lambda i: (0, 0)),
            pl.BlockSpec((8, H), lambda i: (0, 0)),
            pl.BlockSpec((8, 128), lambda i: (0, 0)),
        ],
        out_specs=[
            pl.BlockSpec((_BLK, H), lambda i: (i, 0)),
            pl.BlockSpec((_BLK, H), lambda i: (i, 0)),
            pl.BlockSpec((2, _BLK, H), lambda i: (0, i, 0)),
        ],
        out_shape=[
            jax.ShapeDtypeStruct((N, H), jnp.float32),
            jax.ShapeDtypeStruct((N, H), jnp.float32),
            jax.ShapeDtypeStruct((2, N, H), jnp.float32),
        ],
    )(h0, dn, w1, b1, g1, be1, w2, b2, lng, lnb, t)


def _tc_post_body(x1_ref, hr_ref, dn_ref, w1_ref, b1_ref, g1_ref, be1_ref,
                  w2_ref, b2_ref, lng_ref, lnb_ref, wl_ref, bl_ref,
                  logit_ref):
    den = dn_ref[0]
    num = dn_ref[1]
    out1 = num / (den + 1e-16) + hr_ref[...]
    hh = jnp.dot(out1, w1_ref[...], preferred_element_type=jnp.float32)
    hh = hh + b1_ref[0:1, :]
    hh = jnp.maximum(_ln(hh, g1_ref[0:1, :], be1_ref[0:1, :]), 0.0)
    h2 = jnp.dot(hh, w2_ref[...], preferred_element_type=jnp.float32)
    h2 = h2 + b2_ref[0:1, :]
    x2 = x1_ref[...] + h2
    hf = jnp.maximum(_ln(x2, lng_ref[0:1, :], lnb_ref[0:1, :]), 0.0)
    logit_ref[...] = (
        jnp.dot(hf, wl_ref[...], preferred_element_type=jnp.float32)
        + bl_ref[0, 0]
    )


def _tc_post(x1, hr, dn, w1, b1, g1, be1, w2, b2, lng, lnb, wl, bl):
    return pl.pallas_call(
        _tc_post_body,
        grid=(_GRID,),
        in_specs=[
            pl.BlockSpec((_BLK, H), lambda i: (i, 0)),
            pl.BlockSpec((_BLK, H), lambda i: (i, 0)),
            pl.BlockSpec((2, _BLK, H), lambda i: (0, i, 0)),
            pl.BlockSpec((H, 2 * H), lambda i: (0, 0)),
            pl.BlockSpec((8, 2 * H), lambda i: (0, 0)),
            pl.BlockSpec((8, 2 * H), lambda i: (0, 0)),
            pl.BlockSpec((8, 2 * H), lambda i: (0, 0)),
            pl.BlockSpec((2 * H, H), lambda i: (0, 0)),
            pl.BlockSpec((8, H), lambda i: (0, 0)),
            pl.BlockSpec((8, H), lambda i: (0, 0)),
            pl.BlockSpec((8, H), lambda i: (0, 0)),
            pl.BlockSpec((H, 1), lambda i: (0, 0)),
            pl.BlockSpec((8, 128), lambda i: (0, 0)),
        ],
        out_specs=pl.BlockSpec((_BLK, 1), lambda i: (i, 0)),
        out_shape=jax.ShapeDtypeStruct((N, 1), jnp.float32),
    )(x1, hr, dn, w1, b1, g1, be1, w2, b2, lng, lnb, wl, bl)


def _sc_conv(tab, tsrc2, dst2):
    @functools.partial(
        pl.kernel,
        out_type=jax.ShapeDtypeStruct((2 * N, H), jnp.float32),
        mesh=_MESH,
        scratch_types=[
            pltpu.VMEM((_CNW, _CW), jnp.int32),
            pltpu.VMEM((_CNW, _CW), jnp.int32),
            pltpu.VMEM((_CW, H), jnp.float32),
            pltpu.VMEM((_CW, H), jnp.float32),
            pltpu.VMEM((_CW, H), jnp.float32),
            pltpu.VMEM((_CW, H), jnp.float32),
            pltpu.VMEM((_RZB, H), jnp.float32),
            pltpu.VMEM_SHARED((N, H), jnp.float32),
            pltpu.SemaphoreType.DMA,
            pltpu.SemaphoreType.DMA,
            pltpu.SemaphoreType.DMA,
            pltpu.SemaphoreType.DMA,
            pltpu.SemaphoreType.DMA,
            pltpu.SemaphoreType.DMA,
            pltpu.SemaphoreType.DMA,
            pltpu.SemaphoreType.DMA,
        ],
        compiler_params=_SC_PARAMS,
    )
    def k(tab_h, src_h, dst_h, out_h, src_v, dst_v, r0_v, r1_v, r2_v, r3_v,
          zb_v, acc_sh, sg0, sg1, sg2, sg3, ss0, ss1, ss2, ss3):
        bufs = (r0_v, r1_v, r2_v, r3_v)
        sgs = (sg0, sg1, sg2, sg3)
        sss = (ss0, ss1, ss2, ss3)
        c = lax.axis_index("c")
        s = lax.axis_index("s")
        zero = jnp.zeros((16,), jnp.float32)

        @pl.loop(0, _RZB)
        def _(i):
            zb_v[i, pl.ds(0, 16)] = zero
            zb_v[i, pl.ds(16, 16)] = zero
            zb_v[i, pl.ds(32, 16)] = zero
            zb_v[i, pl.ds(48, 16)] = zero

        rb = s * _RB
        ncp = jnp.where(s == 15, 5, 8)

        def zcp(i, carry):
            pltpu.sync_copy(zb_v, acc_sh.at[pl.ds(rb + i * _RZB, _RZB)])
            return carry

        lax.fori_loop(0, ncp, zcp, 0)

        pltpu.sync_copy(src_h.at[pl.ds((c * 16 + s) * _CNW, _CNW)], src_v)
        pltpu.sync_copy(dst_h.at[pl.ds(s * _CNW, _CNW)], dst_v)

        plsc.subcore_barrier()

        for b in range(4):
            pltpu.async_copy(tab_h.at[src_v.at[b]], bufs[b], sgs[b])

        @pl.loop(0, _CNW - 4, step=4)
        def _(j):
            for b in range(4):
                pltpu.make_async_copy(
                    tab_h.at[src_v.at[j + b]], bufs[b], sgs[b]).wait()
                pltpu.async_copy(
                    bufs[b], acc_sh.at[dst_v.at[j + b]], sss[b], add=True)
            for b in range(4):
                pltpu.make_async_copy(
                    bufs[b], acc_sh.at[dst_v.at[j + b]], sss[b]).wait()
                pltpu.async_copy(
                    tab_h.at[src_v.at[j + 4 + b]], bufs[b], sgs[b])

        j0 = _CNW - 4
        for b in range(4):
            pltpu.make_async_copy(
                tab_h.at[src_v.at[j0 + b]], bufs[b], sgs[b]).wait()
            pltpu.async_copy(
                bufs[b], acc_sh.at[dst_v.at[j0 + b]], sss[b], add=True)
        for b in range(4):
            pltpu.make_async_copy(
                bufs[b], acc_sh.at[dst_v.at[j0 + b]], sss[b]).wait()

        plsc.subcore_barrier()

        def wcp(i, carry):
            pltpu.sync_copy(
                acc_sh.at[pl.ds(rb + i * _RZB, _RZB)],
                out_h.at[pl.ds(c * N + rb + i * _RZB, _RZB)],
            )
            return carry

        lax.fori_loop(0, ncp, wcp, 0)

    return k(tab, tsrc2, dst2)


def _sc_nmax(lg, src3, dst3):
    @functools.partial(
        pl.kernel,
        out_type=jax.ShapeDtypeStruct((2 * N, 16), jnp.float32),
        mesh=_MESH,
        scratch_types=[
            pltpu.VMEM((_NNW, _NW), jnp.int32),
            pltpu.VMEM((_NNW, _NW), jnp.int32),
            pltpu.VMEM((N,), jnp.float32),
            pltpu.VMEM((_NW, 16), jnp.float32),
            pltpu.VMEM((_RZB, 16), jnp.float32),
            pltpu.VMEM_SHARED((N, 16), jnp.float32),
        ],
        compiler_params=_SC_PARAMS_NOLAYOUT,
    )
    def k(lg_h, src_h, dst_h, out_h, src_v, dst_v, lg_v, col_v, zb_v,
          cnt_sh):
        c = lax.axis_index("c")
        s = lax.axis_index("s")
        zero = jnp.zeros((16,), jnp.float32)
        lanes = lax.iota(jnp.int32, 16)
        zlane = jnp.zeros((16,), jnp.int32)

        @pl.loop(0, _RZB)
        def _(i):
            zb_v[i, pl.ds(0, 16)] = zero

        @pl.loop(0, _NW)
        def _(i):
            col_v[i, pl.ds(0, 16)] = zero

        rb = s * _RB
        ncp = jnp.where(s == 15, 5, 8)

        def zcp(i, carry):
            pltpu.sync_copy(zb_v, cnt_sh.at[pl.ds(rb + i * _RZB, _RZB)])
            return carry

        lax.fori_loop(0, ncp, zcp, 0)

        pltpu.sync_copy(lg_h, lg_v)
        pltpu.sync_copy(src_h.at[pl.ds((c * 16 + s) * _NNW, _NNW)], src_v)
        pltpu.sync_copy(dst_h.at[pl.ds((c * 16 + s) * _NNW, _NNW)], dst_v)

        plsc.subcore_barrier()

        @pl.loop(0, _NNW)
        def _(j):
            @pl.loop(0, _NW // 16)
            def _(g):
                sv = src_v[j, pl.ds(g * 16, 16)]
                dv = dst_v[j, pl.ds(g * 16, 16)]
                ls = plsc.load_gather(lg_v, [sv])
                ld = plsc.load_gather(lg_v, [dv])
                ind = jnp.where(ls > ld, 1.0, 0.0).astype(jnp.float32)
                plsc.store_scatter(col_v, [g * 16 + lanes, zlane], ind)

            pltpu.sync_copy(col_v, cnt_sh.at[dst_v.at[j]], add=True)

        plsc.subcore_barrier()

        def wcp(i, carry):
            pltpu.sync_copy(
                cnt_sh.at[pl.ds(rb + i * _RZB, _RZB)],
                out_h.at[pl.ds(c * N + rb + i * _RZB, _RZB)],
            )
            return carry

        lax.fori_loop(0, ncp, wcp, 0)

    return k(lg, src3, dst3)


def _tc_final_body(logit_ref, cnt_ref, out_ref):
    tot = cnt_ref[0, :, 0:1] + cnt_ref[1, :, 0:1]
    out_ref[...] = jnp.where(tot > 0.0, 0.0, 1.0)


def _tc_final(logits, cnt):
    return pl.pallas_call(
        _tc_final_body,
        grid=(_GRID,),
        in_specs=[
            pl.BlockSpec((_BLK, 1), lambda i: (i, 0)),
            pl.BlockSpec((2, _BLK, 16), lambda i: (0, i, 0)),
        ],
        out_specs=pl.BlockSpec((_BLK, 1), lambda i: (i, 0)),
        out_shape=jax.ShapeDtypeStruct((N, 1), jnp.float32),
    )(logits, cnt)


def kernel(x, edge_index, We, be, t0, W1_0, b1_0, g1_0, beta1_0, W2_0, b2_0,
           t1, W1_1, b1_1, g1_1, beta1_1, W2_1, b2_1, ln1_g, ln1_b, ln0_g,
           ln0_b, Wl, bl):
    f32 = jnp.float32

    def r2(v, d):
        return jnp.broadcast_to(v.reshape(1, d).astype(f32), (8, d))

    src = edge_index[0]
    dst = edge_index[1]
    tsrc2 = jnp.concatenate([src, src + N]).reshape(2 * E // _CW, _CW)
    dst2 = dst.reshape(E // _CW, _CW)
    pad = jnp.zeros((_E2 - E,), jnp.int32)
    src3 = jnp.concatenate([src, pad]).reshape(_E2 // _NW, _NW)
    dst3 = jnp.concatenate([dst, pad]).reshape(_E2 // _NW, _NW)

    t0b = jnp.broadcast_to(t0.reshape(1, 1).astype(f32), (8, 128))
    t1b = jnp.broadcast_to(t1.reshape(1, 1).astype(f32), (8, 128))

    h0, tab0 = _tc_pre(x, We, r2(be, H), t0b)
    dn0 = _sc_conv(tab0.reshape(2 * N, H), tsrc2, dst2).reshape(2, N, H)
    x1, hr, tab1 = _tc_mid(h0, dn0, W1_0, r2(b1_0, 2 * H), r2(g1_0, 2 * H),
                           r2(beta1_0, 2 * H), W2_0, r2(b2_0, H),
                           r2(ln1_g, H), r2(ln1_b, H), t1b)
    dn1 = _sc_conv(tab1.reshape(2 * N, H), tsrc2, dst2).reshape(2, N, H)
    logits = _tc_post(x1, hr, dn1, W1_1, r2(b1_1, 2 * H), r2(g1_1, 2 * H),
                      r2(beta1_1, 2 * H), W2_1, r2(b2_1, H), r2(ln0_g, H),
                      r2(ln0_b, H), Wl,
                      jnp.broadcast_to(bl.reshape(1, 1).astype(f32),
                                       (8, 128)))
    cnt = _sc_nmax(logits.reshape(N), src3, dst3).reshape(2, N, 16)
    out = _tc_final(logits, cnt)
    return (out, logits)


# speedup vs baseline: 26.7126x; 1.0301x over previous
import functools

import jax
import jax.numpy as jnp
from jax import lax
from jax.experimental import pallas as pl
from jax.experimental.pallas import tpu as pltpu
from jax.experimental.pallas import tpu_sc as plsc

N = 10000
E = 320000
D = 128
H = 64

_BLK = 1000
_GRID = N // _BLK

_CW = 125
_CNW = E // 16 // _CW
_RB = 640
_NBUF = 5
_RZB = 80

_NW = 80
_NNW = 128
_E2 = 32 * _NNW * _NW

_MESH = plsc.VectorSubcoreMesh(core_axis_name="c", subcore_axis_name="s")
_SC_PARAMS = pltpu.CompilerParams(use_tc_tiling_on_sc=False)
_SC_PARAMS_NOLAYOUT = pltpu.CompilerParams(
    use_tc_tiling_on_sc=False, needs_layout_passes=False)


def _ln(h, g, b):
    mu = jnp.mean(h, axis=-1, keepdims=True)
    var = jnp.mean((h - mu) ** 2, axis=-1, keepdims=True)
    return (h - mu) / jnp.sqrt(var + 1e-5) * g + b


def _tab(h, t):
    m = jnp.maximum(h, 0.0) + 1e-7
    a = jnp.exp(t * m)
    return a, m * a


def _tc_pre_body(x_ref, we_ref, be_ref, t_ref, h0_ref, tab_ref):
    h0 = jnp.dot(x_ref[...], we_ref[...], preferred_element_type=jnp.float32)
    h0 = h0 + be_ref[0:1, :]
    h0_ref[...] = h0
    a, b = _tab(h0, t_ref[0, 0])
    tab_ref[0] = a
    tab_ref[1] = b


def _tc_pre(x, we, be, t):
    return pl.pallas_call(
        _tc_pre_body,
        grid=(_GRID,),
        in_specs=[
            pl.BlockSpec((_BLK, D), lambda i: (i, 0)),
            pl.BlockSpec((D, H), lambda i: (0, 0)),
            pl.BlockSpec((8, H), lambda i: (0, 0)),
            pl.BlockSpec((8, 128), lambda i: (0, 0)),
        ],
        out_specs=[
            pl.BlockSpec((_BLK, H), lambda i: (i, 0)),
            pl.BlockSpec((2, _BLK, H), lambda i: (0, i, 0)),
        ],
        out_shape=[
            jax.ShapeDtypeStruct((N, H), jnp.float32),
            jax.ShapeDtypeStruct((2, N, H), jnp.float32),
        ],
    )(x, we, be, t)


def _tc_mid_body(h0_ref, dn_ref, w1_ref, b1_ref, g1_ref, be1_ref, w2_ref,
                 b2_ref, lng_ref, lnb_ref, t_ref, x1_ref, hr_ref, tab_ref):
    den = dn_ref[0]
    num = dn_ref[1]
    out0 = num / (den + 1e-16) + h0_ref[...]
    hh = jnp.dot(out0, w1_ref[...], preferred_element_type=jnp.float32)
    hh = hh + b1_ref[0:1, :]
    hh = jnp.maximum(_ln(hh, g1_ref[0:1, :], be1_ref[0:1, :]), 0.0)
    x1 = jnp.dot(hh, w2_ref[...], preferred_element_type=jnp.float32)
    x1 = x1 + b2_ref[0:1, :]
    x1_ref[...] = x1
    hr = jnp.maximum(_ln(x1, lng_ref[0:1, :], lnb_ref[0:1, :]), 0.0)
    hr_ref[...] = hr
    a, b = _tab(hr, t_ref[0, 0])
    tab_ref[0] = a
    tab_ref[1] = b


def _tc_mid(h0, dn, w1, b1, g1, be1, w2, b2, lng, lnb, t):
    return pl.pallas_call(
        _tc_mid_body,
        grid=(_GRID,),
        in_specs=[
            pl.BlockSpec((_BLK, H), lambda i: (i, 0)),
            pl.BlockSpec((2, _BLK, H), lambda i: (0, i, 0)),
            pl.BlockSpec((H, 2 * H), lambda i: (0, 0)),
            pl.BlockSpec((8, 2 * H), lambda i: (0, 0)),
            pl.BlockSpec((8, 2 * H), lambda i: (0, 0)),
            pl.BlockSpec((8, 2 * H), lambda i: (0, 0)),
            pl.BlockSpec((2 * H, H), lambda i: (0, 0)),
            pl.BlockSpec((8, H), lambda i: (0, 0)),
            pl.BlockSpec((8, H), lambda i: (0, 0)),
            pl.BlockSpec((8, H), lambda i: (0, 0)),
            pl.BlockSpec((8, 128), lambda i: (0, 0)),
        ],
        out_specs=[
            pl.BlockSpec((_BLK, H), lambda i: (i, 0)),
            pl.BlockSpec((_BLK, H), lambda i: (i, 0)),
            pl.BlockSpec((2, _BLK, H), lambda i: (0, i, 0)),
        ],
        out_shape=[
            jax.ShapeDtypeStruct((N, H), jnp.float32),
            jax.ShapeDtypeStruct((N, H), jnp.float32),
            jax.ShapeDtypeStruct((2, N, H), jnp.float32),
        ],
    )(h0, dn, w1, b1, g1, be1, w2, b2, lng, lnb, t)


def _tc_post_body(x1_ref, hr_ref, dn_ref, w1_ref, b1_ref, g1_ref, be1_ref,
                  w2_ref, b2_ref, lng_ref, lnb_ref, wl_ref, bl_ref,
                  logit_ref):
    den = dn_ref[0]
    num = dn_ref[1]
    out1 = num / (den + 1e-16) + hr_ref[...]
    hh = jnp.dot(out1, w1_ref[...], preferred_element_type=jnp.float32)
    hh = hh + b1_ref[0:1, :]
    hh = jnp.maximum(_ln(hh, g1_ref[0:1, :], be1_ref[0:1, :]), 0.0)
    h2 = jnp.dot(hh, w2_ref[...], preferred_element_type=jnp.float32)
    h2 = h2 + b2_ref[0:1, :]
    x2 = x1_ref[...] + h2
    hf = jnp.maximum(_ln(x2, lng_ref[0:1, :], lnb_ref[0:1, :]), 0.0)
    logit_ref[...] = (
        jnp.dot(hf, wl_ref[...], preferred_element_type=jnp.float32)
        + bl_ref[0, 0]
    )


def _tc_post(x1, hr, dn, w1, b1, g1, be1, w2, b2, lng, lnb, wl, bl):
    return pl.pallas_call(
        _tc_post_body,
        grid=(_GRID,),
        in_specs=[
            pl.BlockSpec((_BLK, H), lambda i: (i, 0)),
            pl.BlockSpec((_BLK, H), lambda i: (i, 0)),
            pl.BlockSpec((2, _BLK, H), lambda i: (0, i, 0)),
            pl.BlockSpec((H, 2 * H), lambda i: (0, 0)),
            pl.BlockSpec((8, 2 * H), lambda i: (0, 0)),
            pl.BlockSpec((8, 2 * H), lambda i: (0, 0)),
            pl.BlockSpec((8, 2 * H), lambda i: (0, 0)),
            pl.BlockSpec((2 * H, H), lambda i: (0, 0)),
            pl.BlockSpec((8, H), lambda i: (0, 0)),
            pl.BlockSpec((8, H), lambda i: (0, 0)),
            pl.BlockSpec((8, H), lambda i: (0, 0)),
            pl.BlockSpec((H, 1), lambda i: (0, 0)),
            pl.BlockSpec((8, 128), lambda i: (0, 0)),
        ],
        out_specs=pl.BlockSpec((_BLK, 1), lambda i: (i, 0)),
        out_shape=jax.ShapeDtypeStruct((N, 1), jnp.float32),
    )(x1, hr, dn, w1, b1, g1, be1, w2, b2, lng, lnb, wl, bl)


def _sc_conv(tab, tsrc2, dst2):
    @functools.partial(
        pl.kernel,
        out_type=jax.ShapeDtypeStruct((2 * N, H), jnp.float32),
        mesh=_MESH,
        scratch_types=[
            pltpu.VMEM((_CNW, _CW), jnp.int32),
            pltpu.VMEM((_CNW, _CW), jnp.int32),
        ]
        + [pltpu.VMEM((_CW, H), jnp.float32)] * _NBUF
        + [
            pltpu.VMEM((_RZB, H), jnp.float32),
            pltpu.VMEM_SHARED((N, H), jnp.float32),
        ]
        + [pltpu.SemaphoreType.DMA] * (2 * _NBUF),
        compiler_params=_SC_PARAMS,
    )
    def k(tab_h, src_h, dst_h, out_h, src_v, dst_v, *rest):
        bufs = rest[:_NBUF]
        zb_v = rest[_NBUF]
        acc_sh = rest[_NBUF + 1]
        sgs = rest[_NBUF + 2:2 * _NBUF + 2]
        sss = rest[2 * _NBUF + 2:]
        c = lax.axis_index("c")
        s = lax.axis_index("s")
        zero = jnp.zeros((16,), jnp.float32)

        @pl.loop(0, _RZB)
        def _(i):
            zb_v[i, pl.ds(0, 16)] = zero
            zb_v[i, pl.ds(16, 16)] = zero
            zb_v[i, pl.ds(32, 16)] = zero
            zb_v[i, pl.ds(48, 16)] = zero

        rb = s * _RB
        ncp = jnp.where(s == 15, 5, 8)

        def zcp(i, carry):
            pltpu.sync_copy(zb_v, acc_sh.at[pl.ds(rb + i * _RZB, _RZB)])
            return carry

        lax.fori_loop(0, ncp, zcp, 0)

        pltpu.sync_copy(src_h.at[pl.ds((c * 16 + s) * _CNW, _CNW)], src_v)
        pltpu.sync_copy(dst_h.at[pl.ds(s * _CNW, _CNW)], dst_v)

        plsc.subcore_barrier()

        for b in range(_NBUF):
            pltpu.async_copy(tab_h.at[src_v.at[b]], bufs[b], sgs[b])

        @pl.loop(0, _CNW - _NBUF, step=_NBUF)
        def _(j):
            for b in range(_NBUF):
                pltpu.make_async_copy(
                    tab_h.at[src_v.at[j + b]], bufs[b], sgs[b]).wait()
                pltpu.async_copy(
                    bufs[b], acc_sh.at[dst_v.at[j + b]], sss[b], add=True)
            for b in range(_NBUF):
                pltpu.make_async_copy(
                    bufs[b], acc_sh.at[dst_v.at[j + b]], sss[b]).wait()
                pltpu.async_copy(
                    tab_h.at[src_v.at[j + _NBUF + b]], bufs[b], sgs[b])

        j0 = _CNW - _NBUF
        for b in range(_NBUF):
            pltpu.make_async_copy(
                tab_h.at[src_v.at[j0 + b]], bufs[b], sgs[b]).wait()
            pltpu.async_copy(
                bufs[b], acc_sh.at[dst_v.at[j0 + b]], sss[b], add=True)
        for b in range(_NBUF):
            pltpu.make_async_copy(
                bufs[b], acc_sh.at[dst_v.at[j0 + b]], sss[b]).wait()

        plsc.subcore_barrier()

        def wcp(i, carry):
            pltpu.sync_copy(
                acc_sh.at[pl.ds(rb + i * _RZB, _RZB)],
                out_h.at[pl.ds(c * N + rb + i * _RZB, _RZB)],
            )
            return carry

        lax.fori_loop(0, ncp, wcp, 0)

    return k(tab, tsrc2, dst2)


def _sc_nmax(lg, src3, dst3):
    @functools.partial(
        pl.kernel,
        out_type=jax.ShapeDtypeStruct((2 * N, 16), jnp.float32),
        mesh=_MESH,
        scratch_types=[
            pltpu.VMEM((_NNW, _NW), jnp.int32),
            pltpu.VMEM((_NNW, _NW), jnp.int32),
            pltpu.VMEM((N,), jnp.float32),
            pltpu.VMEM((_NW, 16), jnp.float32),
            pltpu.VMEM((_NW, 16), jnp.float32),
            pltpu.VMEM((_RZB, 16), jnp.float32),
            pltpu.VMEM_SHARED((N, 16), jnp.float32),
            pltpu.SemaphoreType.DMA,
            pltpu.SemaphoreType.DMA,
        ],
        compiler_params=_SC_PARAMS_NOLAYOUT,
    )
    def k(lg_h, src_h, dst_h, out_h, src_v, dst_v, lg_v, colA, colB, zb_v,
          cnt_sh, sca, scb):
        c = lax.axis_index("c")
        s = lax.axis_index("s")
        zero = jnp.zeros((16,), jnp.float32)
        lanes = lax.iota(jnp.int32, 16)
        zlane = jnp.zeros((16,), jnp.int32)

        @pl.loop(0, _RZB)
        def _(i):
            zb_v[i, pl.ds(0, 16)] = zero

        @pl.loop(0, _NW)
        def _(i):
            colA[i, pl.ds(0, 16)] = zero
            colB[i, pl.ds(0, 16)] = zero

        rb = s * _RB
        ncp = jnp.where(s == 15, 5, 8)

        def zcp(i, carry):
            pltpu.sync_copy(zb_v, cnt_sh.at[pl.ds(rb + i * _RZB, _RZB)])
            return carry

        lax.fori_loop(0, ncp, zcp, 0)

        pltpu.sync_copy(lg_h, lg_v)
        pltpu.sync_copy(src_h.at[pl.ds((c * 16 + s) * _NNW, _NNW)], src_v)
        pltpu.sync_copy(dst_h.at[pl.ds((c * 16 + s) * _NNW, _NNW)], dst_v)

        plsc.subcore_barrier()

        def win(jj, cb):
            @pl.loop(0, _NW // 16)
            def _(g):
                sv = src_v[jj, pl.ds(g * 16, 16)]
                dv = dst_v[jj, pl.ds(g * 16, 16)]
                ls = plsc.load_gather(lg_v, [sv])
                ld = plsc.load_gather(lg_v, [dv])
                ind = jnp.where(ls > ld, 1.0, 0.0).astype(jnp.float32)
                plsc.store_scatter(cb, [g * 16 + lanes, zlane], ind)

        win(0, colA)
        pltpu.async_copy(colA, cnt_sh.at[dst_v.at[0]], sca, add=True)
        win(1, colB)
        pltpu.async_copy(colB, cnt_sh.at[dst_v.at[1]], scb, add=True)

        @pl.loop(2, _NNW, step=2)
        def _(j):
            pltpu.make_async_copy(
                colA, cnt_sh.at[dst_v.at[j - 2]], sca).wait()
            win(j, colA)
            pltpu.async_copy(colA, cnt_sh.at[dst_v.at[j]], sca, add=True)
            pltpu.make_async_copy(
                colB, cnt_sh.at[dst_v.at[j - 1]], scb).wait()
            win(j + 1, colB)
            pltpu.async_copy(colB, cnt_sh.at[dst_v.at[j + 1]], scb, add=True)

        pltpu.make_async_copy(
            colA, cnt_sh.at[dst_v.at[_NNW - 2]], sca).wait()
        pltpu.make_async_copy(
            colB, cnt_sh.at[dst_v.at[_NNW - 1]], scb).wait()

        plsc.subcore_barrier()

        def wcp(i, carry):
            pltpu.sync_copy(
                cnt_sh.at[pl.ds(rb + i * _RZB, _RZB)],
                out_h.at[pl.ds(c * N + rb + i * _RZB, _RZB)],
            )
            return carry

        lax.fori_loop(0, ncp, wcp, 0)

    return k(lg, src3, dst3)


def _tc_final_body(logit_ref, cnt_ref, out_ref):
    tot = cnt_ref[0, :, 0:1] + cnt_ref[1, :, 0:1]
    out_ref[...] = jnp.where(tot > 0.0, 0.0, 1.0)


def _tc_final(logits, cnt):
    return pl.pallas_call(
        _tc_final_body,
        grid=(_GRID,),
        in_specs=[
            pl.BlockSpec((_BLK, 1), lambda i: (i, 0)),
            pl.BlockSpec((2, _BLK, 16), lambda i: (0, i, 0)),
        ],
        out_specs=pl.BlockSpec((_BLK, 1), lambda i: (i, 0)),
        out_shape=jax.ShapeDtypeStruct((N, 1), jnp.float32),
    )(logits, cnt)


def kernel(x, edge_index, We, be, t0, W1_0, b1_0, g1_0, beta1_0, W2_0, b2_0,
           t1, W1_1, b1_1, g1_1, beta1_1, W2_1, b2_1, ln1_g, ln1_b, ln0_g,
           ln0_b, Wl, bl):
    f32 = jnp.float32

    def r2(v, d):
        return jnp.broadcast_to(v.reshape(1, d).astype(f32), (8, d))

    src = edge_index[0]
    dst = edge_index[1]
    tsrc2 = jnp.concatenate([src, src + N]).reshape(2 * E // _CW, _CW)
    dst2 = dst.reshape(E // _CW, _CW)
    pad = jnp.zeros((_E2 - E,), jnp.int32)
    src3 = jnp.concatenate([src, pad]).reshape(_E2 // _NW, _NW)
    dst3 = jnp.concatenate([dst, pad]).reshape(_E2 // _NW, _NW)

    t0b = jnp.broadcast_to(t0.reshape(1, 1).astype(f32), (8, 128))
    t1b = jnp.broadcast_to(t1.reshape(1, 1).astype(f32), (8, 128))

    h0, tab0 = _tc_pre(x, We, r2(be, H), t0b)
    dn0 = _sc_conv(tab0.reshape(2 * N, H), tsrc2, dst2).reshape(2, N, H)
    x1, hr, tab1 = _tc_mid(h0, dn0, W1_0, r2(b1_0, 2 * H), r2(g1_0, 2 * H),
                           r2(beta1_0, 2 * H), W2_0, r2(b2_0, H),
                           r2(ln1_g, H), r2(ln1_b, H), t1b)
    dn1 = _sc_conv(tab1.reshape(2 * N, H), tsrc2, dst2).reshape(2, N, H)
    logits = _tc_post(x1, hr, dn1, W1_1, r2(b1_1, 2 * H), r2(g1_1, 2 * H),
                      r2(beta1_1, 2 * H), W2_1, r2(b2_1, H), r2(ln0_g, H),
                      r2(ln0_b, H), Wl,
                      jnp.broadcast_to(bl.reshape(1, 1).astype(f32),
                                       (8, 128)))
    cnt = _sc_nmax(logits.reshape(N), src3, dst3).reshape(2, N, 16)
    out = _tc_final(logits, cnt)
    return (out, logits)
